# baseline (speedup 1.0000x reference)
import functools

import jax
import jax.numpy as jnp
from jax import lax
from jax.experimental import pallas as pl
from jax.experimental.pallas import tpu as pltpu
from jax.experimental.pallas import tpu_sc as plsc

NE = 8192
ED = 32
BR = 512
CH = 4096
N_ROWS = 32768


def _round_bf16_bits(x):
    bits = jax.lax.bitcast_convert_type(x, jnp.uint32)
    add = ((bits >> 16) & jnp.uint32(1)) + jnp.uint32(0x7FFF)
    rbits = (bits + add) & jnp.uint32(0xFFFF0000)
    return jax.lax.bitcast_convert_type(rbits, jnp.float32)


def _chunk_argmin(s2, z2, w2_ref, lo):
    acc_v = None
    acc_k = None
    for k in range(CH // 128):
        sl = s2[:, lo + k * 128: lo + (k + 1) * 128]
        d = (z2 + w2_ref[:, lo + k * 128: lo + (k + 1) * 128]) - sl
        if acc_v is None:
            acc_v = d
            acc_k = jnp.zeros(d.shape, jnp.int32)
        else:
            upd = d < acc_v
            acc_v = jnp.where(upd, d, acc_v)
            acc_k = jnp.where(upd, k, acc_k)
    m = jnp.min(acc_v, axis=1, keepdims=True)
    lane = jax.lax.broadcasted_iota(jnp.int32, acc_v.shape, 1)
    j = acc_k * 128 + lane + lo
    idx = jnp.min(jnp.where(acc_v == m, j, NE), axis=1, keepdims=True)
    return m, idx


def _vq_block(z_ref, w_ref, z2_ref, w2_ref, codes_ref, loss_ref):
    i = pl.program_id(0)
    zb2 = z_ref[...]
    w = w_ref[...]
    s2 = jax.lax.dot_general(zb2, w, (((1,), (1,)), ((), ())),
                             preferred_element_type=jnp.float32)
    z2 = z2_ref[...]
    m0, i0 = _chunk_argmin(s2, z2, w2_ref, 0)
    m1, i1 = _chunk_argmin(s2, z2, w2_ref, CH)
    take1 = m1 < _round_bf16_bits(m0)
    codes = jnp.where(take1, i1, i0)
    sel_min = jnp.where(take1, m1, m0)
    codes_ref[...] = codes.reshape(1, 1, BR)
    lsum = jnp.sum(sel_min)

    @pl.when(i == 0)
    def _():
        loss_ref[0, 0] = 0.0

    loss_ref[0, 0] += lsum


_SC_INFO = plsc.get_sparse_core_info()
_NW = _SC_INFO.num_cores * _SC_INFO.num_subcores
_B_PER_W = (N_ROWS // 2) // _NW
_B_CHUNK = _B_PER_W
_PD = 128


def _sc_gather(table_hbm, idx_hbm, out_hbm, idx_v, rows_v, sem):
    wid = lax.axis_index("s") * _SC_INFO.num_cores + lax.axis_index("c")
    base = wid * _B_PER_W
    for k in range(_B_PER_W // _B_CHUNK):
        off = base + k * _B_CHUNK
        pltpu.sync_copy(idx_hbm.at[pl.ds(off, _B_CHUNK)], idx_v)
        pltpu.async_copy(table_hbm.at[idx_v], rows_v, sem).wait()
        pltpu.sync_copy(rows_v, out_hbm.at[pl.ds(off, _B_CHUNK)])


def _tc_half(z2x_half, W, z2_half, w2):
    grid = z2x_half.shape[0] // BR
    codes3, loss = pl.pallas_call(
        _vq_block,
        grid=(grid,),
        in_specs=[
            pl.BlockSpec((BR, ED), lambda i: (i, 0)),
            pl.BlockSpec((NE, ED), lambda i: (0, 0)),
            pl.BlockSpec((BR, 1), lambda i: (i, 0)),
            pl.BlockSpec((1, NE), lambda i: (0, 0)),
        ],
        out_specs=[
            pl.BlockSpec((1, 1, BR), lambda i: (i, 0, 0)),
            pl.BlockSpec(memory_space=pltpu.SMEM),
        ],
        out_shape=[
            jax.ShapeDtypeStruct((grid, 1, BR), jnp.int32),
            jax.ShapeDtypeStruct((1, 1), jnp.float32),
        ],
    )(z2x_half, W, z2_half, w2)
    return codes3.reshape(z2x_half.shape[0]), loss


def kernel(z, W):
    n = z.shape[0] * z.shape[1]
    h = n // 2
    z_flat = z.reshape(n, ED)
    z2 = jnp.sum(z_flat ** 2, axis=1, keepdims=True)
    w2 = jnp.sum(W ** 2, axis=1).reshape(1, NE)
    z2x = 2.0 * z_flat
    w_pad = jnp.pad(W, ((0, 0), (0, _PD - ED)))
    mesh = plsc.VectorSubcoreMesh(core_axis_name="c", subcore_axis_name="s")
    gather = functools.partial(
        pl.kernel, mesh=mesh,
        out_type=jax.ShapeDtypeStruct((h, _PD), jnp.float32),
        scratch_types=[
            pltpu.VMEM((_B_CHUNK,), jnp.int32),
            pltpu.VMEM((_B_CHUNK, _PD), jnp.float32),
            pltpu.SemaphoreType.DMA,
        ],
    )(_sc_gather)

    codes_a, loss_a = _tc_half(z2x[:h], W, z2[:h], w2)
    zq_a = gather(w_pad, codes_a)
    codes_b, loss_b = _tc_half(z2x[h:], W, z2[h:], w2)
    zq_b = gather(w_pad, codes_b)

    zst = jnp.concatenate([zq_a[:, :ED], zq_b[:, :ED]], axis=0)
    codes = jnp.concatenate([codes_a, codes_b])
    vq_loss = 0.25 * (loss_a[0, 0] + loss_b[0, 0]) / (n * ED)
    return (zst.reshape(z.shape), vq_loss, codes.reshape(z.shape[:-1]))


# speedup vs baseline: 1.0941x; 1.0941x over previous
import functools

import jax
import jax.numpy as jnp
from jax import lax
from jax.experimental import pallas as pl
from jax.experimental.pallas import tpu as pltpu
from jax.experimental.pallas import tpu_sc as plsc

NE = 8192
ED = 32
BR = 512
CH = 4096
N_ROWS = 32768


def _round_bf16_bits(x):
    bits = jax.lax.bitcast_convert_type(x, jnp.uint32)
    add = ((bits >> 16) & jnp.uint32(1)) + jnp.uint32(0x7FFF)
    rbits = (bits + add) & jnp.uint32(0xFFFF0000)
    return jax.lax.bitcast_convert_type(rbits, jnp.float32)


def _chunk_argmin(s2, z2, w2_ref, lo):
    acc_v = None
    acc_k = None
    for k in range(CH // 128):
        sl = s2[:, lo + k * 128: lo + (k + 1) * 128]
        d = (z2 + w2_ref[:, lo + k * 128: lo + (k + 1) * 128]) - sl
        if acc_v is None:
            acc_v = d
            acc_k = jnp.zeros(d.shape, jnp.int32)
        else:
            upd = d < acc_v
            acc_v = jnp.where(upd, d, acc_v)
            acc_k = jnp.where(upd, k, acc_k)
    m = jnp.min(acc_v, axis=1, keepdims=True)
    lane = jax.lax.broadcasted_iota(jnp.int32, acc_v.shape, 1)
    j = acc_k * 128 + lane + lo
    idx = jnp.min(jnp.where(acc_v == m, j, NE), axis=1, keepdims=True)
    return m, idx


def _vq_block(z_ref, w_ref, z2_ref, w2_ref, codes_ref, loss_ref):
    i = pl.program_id(0)
    zb2 = z_ref[...] + z_ref[...]
    w = w_ref[...]
    s2 = jax.lax.dot_general(zb2, w, (((1,), (1,)), ((), ())),
                             preferred_element_type=jnp.float32)
    z2 = z2_ref[...]
    m0, i0 = _chunk_argmin(s2, z2, w2_ref, 0)
    m1, i1 = _chunk_argmin(s2, z2, w2_ref, CH)
    take1 = m1 < _round_bf16_bits(m0)
    codes = jnp.where(take1, i1, i0)
    sel_min = jnp.where(take1, m1, m0)
    codes_ref[...] = codes.reshape(1, 1, BR)
    lsum = jnp.sum(sel_min)

    @pl.when(i == 0)
    def _():
        loss_ref[0, 0] = 0.0

    loss_ref[0, 0] += lsum


_SC_INFO = plsc.get_sparse_core_info()
_NW = _SC_INFO.num_cores * _SC_INFO.num_subcores
_B_PER_W = N_ROWS // _NW
_B_CHUNK = _B_PER_W // 2
_PD = 128


def _sc_gather(table_hbm, idx_hbm, out_hbm, idx_v, rows_v, sem):
    wid = lax.axis_index("s") * _SC_INFO.num_cores + lax.axis_index("c")
    base = wid * _B_PER_W
    for k in range(_B_PER_W // _B_CHUNK):
        off = base + k * _B_CHUNK
        pltpu.sync_copy(idx_hbm.at[pl.ds(off, _B_CHUNK)], idx_v)
        pltpu.async_copy(table_hbm.at[idx_v], rows_v, sem).wait()
        pltpu.sync_copy(rows_v, out_hbm.at[pl.ds(off, _B_CHUNK)])


def kernel(z, W):
    n = z.shape[0] * z.shape[1]
    z_flat = z.reshape(n, ED)
    z2 = jnp.sum(z_flat ** 2, axis=1, keepdims=True)
    w2 = jnp.sum(W ** 2, axis=1).reshape(1, NE)
    grid = n // BR
    codes3, loss = pl.pallas_call(
        _vq_block,
        grid=(grid,),
        in_specs=[
            pl.BlockSpec((BR, ED), lambda i: (i, 0)),
            pl.BlockSpec((NE, ED), lambda i: (0, 0)),
            pl.BlockSpec((BR, 1), lambda i: (i, 0)),
            pl.BlockSpec((1, NE), lambda i: (0, 0)),
        ],
        out_specs=[
            pl.BlockSpec((1, 1, BR), lambda i: (i, 0, 0)),
            pl.BlockSpec(memory_space=pltpu.SMEM),
        ],
        out_shape=[
            jax.ShapeDtypeStruct((grid, 1, BR), jnp.int32),
            jax.ShapeDtypeStruct((1, 1), jnp.float32),
        ],
    )(z_flat, W, z2, w2)
    codes = codes3.reshape(n)

    w_pad = jnp.pad(W, ((0, 0), (0, _PD - ED)))
    mesh = plsc.VectorSubcoreMesh(core_axis_name="c", subcore_axis_name="s")
    gather = functools.partial(
        pl.kernel, mesh=mesh,
        out_type=jax.ShapeDtypeStruct((n, _PD), jnp.float32),
        scratch_types=[
            pltpu.VMEM((_B_CHUNK,), jnp.int32),
            pltpu.VMEM((_B_CHUNK, _PD), jnp.float32),
            pltpu.SemaphoreType.DMA,
        ],
    )(_sc_gather)
    zst = gather(w_pad, codes)[:, :ED]

    vq_loss = 0.25 * loss[0, 0] / (n * ED)
    return (zst.reshape(z.shape), vq_loss, codes.reshape(z.shape[:-1]))
